# baseline (speedup 1.0000x reference)
import functools

import jax
import jax.numpy as jnp
from jax import lax
from jax.experimental import pallas as pl
from jax.experimental.pallas import tpu as pltpu
from jax.experimental.pallas import tpu_sc as plsc

N = 10000
K = 32
IN_DIM = 128
OUT_DIM = 128
MID = 32
N_KP = 15
KP_EXTENT = 0.05
NEG_SLOPE = 0.1

NC = 2
NS = 16
NW = NC * NS
NODES_PW = 320
N_PAD = NW * NODES_PW
CH = 4
ROWS = CH * K
NCHUNK = NODES_PW // CH

B3 = 128
NBLK3 = N_PAD // B3


def _leaky(x):
    return jnp.where(x >= 0, x, NEG_SLOPE * x)


def _stage1_body(f_ref, w_ref, b_ref, g_ref, be_ref, x_ref):
    y = jnp.dot(f_ref[...], w_ref[...], preferred_element_type=jnp.float32)
    y = y + b_ref[...]
    m = jnp.mean(y, axis=0, keepdims=True)
    v = jnp.mean((y - m) ** 2, axis=0, keepdims=True)
    y = (y - m) / jnp.sqrt(v + 1e-5)
    x_ref[...] = _leaky(y * g_ref[...] + be_ref[...])


def _stage1(features, W1, b1, g1, be1):
    return pl.pallas_call(
        _stage1_body,
        out_shape=jax.ShapeDtypeStruct((N, MID), jnp.float32),
    )(features, W1, b1.reshape(1, MID), g1.reshape(1, MID), be1.reshape(1, MID))


def _sc_gather_body(feat_hbm, x_hbm, pts_hbm, nidx_hbm,
                    scmax_hbm, nx_hbm, npts_hbm,
                    idx_v, feat_v, x_v, pts_v, mx_v, sem_f, sem_x, sem_p):
    wid = lax.axis_index("s") * NC + lax.axis_index("c")

    def chunk(t, carry):
        node0 = wid * NODES_PW + t * CH
        row0 = node0 * K
        pltpu.sync_copy(nidx_hbm.at[pl.ds(row0, ROWS)], idx_v)
        cf = pltpu.async_copy(feat_hbm.at[idx_v], feat_v, sem_f)
        cx = pltpu.async_copy(x_hbm.at[idx_v], x_v, sem_x)
        cp = pltpu.async_copy(pts_hbm.at[idx_v], pts_v, sem_p)
        cx.wait()
        pltpu.sync_copy(x_v, nx_hbm.at[pl.ds(row0, ROWS)])
        cp.wait()
        pltpu.sync_copy(pts_v, npts_hbm.at[pl.ds(row0, ROWS)])
        cf.wait()
        for i in range(CH):
            def kmax(k, acc):
                return tuple(
                    jnp.maximum(acc[j], feat_v[i * K + k, pl.ds(j * 16, 16)])
                    for j in range(8))
            acc0 = tuple(feat_v[i * K, pl.ds(j * 16, 16)] for j in range(8))
            acc = lax.fori_loop(1, K, kmax, acc0)
            for j in range(8):
                mx_v[i, pl.ds(j * 16, 16)] = acc[j]
        pltpu.sync_copy(mx_v, scmax_hbm.at[pl.ds(node0, CH)])
        return carry

    lax.fori_loop(0, NCHUNK, chunk, 0)


def _stage2(features, x, pts_pad, nidx_flat):
    mesh = plsc.VectorSubcoreMesh(core_axis_name="c", subcore_axis_name="s")
    fn = functools.partial(
        pl.kernel, _sc_gather_body, mesh=mesh,
        compiler_params=pltpu.CompilerParams(use_tc_tiling_on_sc=False),
        out_type=[
            jax.ShapeDtypeStruct((N_PAD, IN_DIM), jnp.float32),
            jax.ShapeDtypeStruct((N_PAD * K, MID), jnp.float32),
            jax.ShapeDtypeStruct((N_PAD * K, 8), jnp.float32),
        ],
        scratch_types=[
            pltpu.VMEM((ROWS,), jnp.int32),
            pltpu.VMEM((ROWS, IN_DIM), jnp.float32),
            pltpu.VMEM((ROWS, MID), jnp.float32),
            pltpu.VMEM((ROWS, 8), jnp.float32),
            pltpu.VMEM((CH, IN_DIM), jnp.float32),
            pltpu.SemaphoreType.DMA,
            pltpu.SemaphoreType.DMA,
            pltpu.SemaphoreType.DMA,
        ],
    )()
    return fn(features, x, pts_pad, nidx_flat)


def _stage3a_body(npts_ref, nx_ref, ctr_ref, kpx_ref, kpy_ref, kpz_ref,
                  kwf_ref, t_ref):
    pts = npts_ref[...]
    rx = pts[:, :, 0:1] - ctr_ref[:, 0:1][:, None, :]
    ry = pts[:, :, 1:2] - ctr_ref[:, 1:2][:, None, :]
    rz = pts[:, :, 2:3] - ctr_ref[:, 2:3][:, None, :]
    nx = nx_ref[...]
    parts = []
    for p in range(N_KP):
        dx = rx - kpx_ref[0, p]
        dy = ry - kpy_ref[0, p]
        dz = rz - kpz_ref[0, p]
        d = jnp.sqrt(dx * dx + dy * dy + dz * dz)
        infl = jnp.maximum(0.0, 1.0 - d * (1.0 / KP_EXTENT))
        parts.append(jnp.sum(infl * nx, axis=1))
    agg = jnp.concatenate(parts, axis=1)
    t = jnp.dot(agg, kwf_ref[...], preferred_element_type=jnp.float32)
    t_ref[...] = _leaky(t)


def _stage3a(npts, nx, ctr, kpx, kpy, kpz, kwf):
    return pl.pallas_call(
        _stage3a_body,
        grid=(NBLK3,),
        in_specs=[
            pl.BlockSpec((B3, K, 8), lambda b: (b, 0, 0)),
            pl.BlockSpec((B3, K, MID), lambda b: (b, 0, 0)),
            pl.BlockSpec((B3, 8), lambda b: (b, 0)),
            pl.BlockSpec((1, 16), lambda b: (0, 0)),
            pl.BlockSpec((1, 16), lambda b: (0, 0)),
            pl.BlockSpec((1, 16), lambda b: (0, 0)),
            pl.BlockSpec((N_KP * MID, MID), lambda b: (0, 0)),
        ],
        out_specs=pl.BlockSpec((B3, MID), lambda b: (b, 0)),
        out_shape=jax.ShapeDtypeStruct((N_PAD, MID), jnp.float32),
    )(npts, nx, ctr, kpx, kpy, kpz, kwf)


def _stage3b_body(t_ref, sm_ref, w2_ref, b2_ref, g2_ref, be2_ref,
                  wsc_ref, bsc_ref, gsc_ref, besc_ref, o_ref):
    u = jnp.dot(t_ref[...], w2_ref[...], preferred_element_type=jnp.float32)
    u = u + b2_ref[...]
    m = jnp.mean(u, axis=0, keepdims=True)
    v = jnp.mean((u - m) ** 2, axis=0, keepdims=True)
    u = _leaky((u - m) / jnp.sqrt(v + 1e-5) * g2_ref[...] + be2_ref[...])
    s = jnp.dot(sm_ref[...], wsc_ref[...], preferred_element_type=jnp.float32)
    s = s + bsc_ref[...]
    ms = jnp.mean(s, axis=0, keepdims=True)
    vs = jnp.mean((s - ms) ** 2, axis=0, keepdims=True)
    s = _leaky((s - ms) / jnp.sqrt(vs + 1e-5) * gsc_ref[...] + besc_ref[...])
    o_ref[...] = _leaky(u + s)


def _stage3b(t, sm, W2, b2, g2, be2, Wsc, bsc, gsc, besc):
    r = lambda a: a.reshape(1, OUT_DIM)
    return pl.pallas_call(
        _stage3b_body,
        out_shape=jax.ShapeDtypeStruct((N, OUT_DIM), jnp.float32),
    )(t, sm, W2, r(b2), r(g2), r(be2), Wsc, r(bsc), r(gsc), r(besc))


def kernel(features, points, neighbors, W1, b1, g1, be1, kp, KW,
           W2, b2, g2, be2, Wsc, bsc, gsc, besc):
    features = features.astype(jnp.float32)
    nidx = jnp.pad(neighbors.astype(jnp.int32), ((0, N_PAD - N), (0, 0)))
    nidx_flat = nidx.reshape(N_PAD * K)
    pts_pad = jnp.pad(points, ((0, 0), (0, 5)))
    ctr = jnp.pad(points, ((0, N_PAD - N), (0, 5)))
    kpx = jnp.pad(kp[:, 0], (0, 1)).reshape(1, 16)
    kpy = jnp.pad(kp[:, 1], (0, 1)).reshape(1, 16)
    kpz = jnp.pad(kp[:, 2], (0, 1)).reshape(1, 16)
    kwf = KW.reshape(N_KP * MID, MID)

    x = _stage1(features, W1, b1, g1, be1)
    scmax, nx_rows, npts_rows = _stage2(features, x, pts_pad, nidx_flat)
    npts = npts_rows.reshape(N_PAD, K, 8)
    nx = nx_rows.reshape(N_PAD, K, MID)
    t = _stage3a(npts, nx, ctr, kpx, kpy, kpz, kwf)
    return _stage3b(t[:N], scmax[:N], W2, b2, g2, be2, Wsc, bsc, gsc, besc)


# speedup vs baseline: 2.7845x; 2.7845x over previous
import functools

import jax
import jax.numpy as jnp
from jax import lax
from jax.experimental import pallas as pl
from jax.experimental.pallas import tpu as pltpu
from jax.experimental.pallas import tpu_sc as plsc

N = 10000
K = 32
IN_DIM = 128
OUT_DIM = 128
MID = 32
N_KP = 15
KP_EXTENT = 0.05
NEG_SLOPE = 0.1

NC = 2
NS = 16
NW = NC * NS
NODES_PW = 320
N_PAD = NW * NODES_PW
CH = 4
ROWS = CH * K
NCHUNK = NODES_PW // CH

B3 = 64
NBLK3 = N_PAD // B3
R3 = B3 * K


def _leaky(x):
    return jnp.where(x >= 0, x, NEG_SLOPE * x)


def _stage1_body(f_ref, w_ref, b_ref, g_ref, be_ref, x_ref):
    y = jnp.dot(f_ref[...], w_ref[...], preferred_element_type=jnp.float32)
    y = y + b_ref[...]
    m = jnp.mean(y, axis=0, keepdims=True)
    v = jnp.mean((y - m) ** 2, axis=0, keepdims=True)
    y = (y - m) / jnp.sqrt(v + 1e-5)
    x_ref[...] = _leaky(y * g_ref[...] + be_ref[...])


def _stage1(features, W1, b1, g1, be1):
    return pl.pallas_call(
        _stage1_body,
        out_shape=jax.ShapeDtypeStruct((N, MID), jnp.float32),
    )(features, W1, b1.reshape(1, MID), g1.reshape(1, MID), be1.reshape(1, MID))


def _sc_gather_body(feat_hbm, x_hbm, pts_hbm, nidx_hbm,
                    scmax_hbm, nx_hbm, npts_hbm,
                    idx_v, feat_v, x_v, pts_v, mx_v, sem_f, sem_x, sem_p):
    wid = lax.axis_index("s") * NC + lax.axis_index("c")

    def chunk(t, carry):
        node0 = wid * NODES_PW + t * CH
        row0 = node0 * K
        pltpu.sync_copy(nidx_hbm.at[pl.ds(row0, ROWS)], idx_v)
        cf = pltpu.async_copy(feat_hbm.at[idx_v], feat_v, sem_f)
        cx = pltpu.async_copy(x_hbm.at[idx_v], x_v, sem_x)
        cp = pltpu.async_copy(pts_hbm.at[idx_v], pts_v, sem_p)
        cx.wait()
        pltpu.sync_copy(x_v, nx_hbm.at[pl.ds(row0, ROWS)])
        cp.wait()
        pltpu.sync_copy(pts_v, npts_hbm.at[pl.ds(row0, ROWS)])
        cf.wait()
        for i in range(CH):
            def kmax(k, acc):
                return tuple(
                    jnp.maximum(acc[j], feat_v[i * K + k, pl.ds(j * 16, 16)])
                    for j in range(8))
            acc0 = tuple(feat_v[i * K, pl.ds(j * 16, 16)] for j in range(8))
            acc = lax.fori_loop(1, K, kmax, acc0)
            for j in range(8):
                mx_v[i, pl.ds(j * 16, 16)] = acc[j]
        pltpu.sync_copy(mx_v, scmax_hbm.at[pl.ds(node0, CH)])
        return carry

    lax.fori_loop(0, NCHUNK, chunk, 0)


def _stage2(features, x, pts_pad, nidx_flat):
    mesh = plsc.VectorSubcoreMesh(core_axis_name="c", subcore_axis_name="s")
    fn = functools.partial(
        pl.kernel, _sc_gather_body, mesh=mesh,
        compiler_params=pltpu.CompilerParams(use_tc_tiling_on_sc=False),
        out_type=[
            jax.ShapeDtypeStruct((N_PAD, IN_DIM), jnp.float32),
            jax.ShapeDtypeStruct((N_PAD * K, MID), jnp.float32),
            jax.ShapeDtypeStruct((N_PAD * K, 8), jnp.float32),
        ],
        scratch_types=[
            pltpu.VMEM((ROWS,), jnp.int32),
            pltpu.VMEM((ROWS, IN_DIM), jnp.float32),
            pltpu.VMEM((ROWS, MID), jnp.float32),
            pltpu.VMEM((ROWS, 8), jnp.float32),
            pltpu.VMEM((CH, IN_DIM), jnp.float32),
            pltpu.SemaphoreType.DMA,
            pltpu.SemaphoreType.DMA,
            pltpu.SemaphoreType.DMA,
        ],
    )()
    return fn(features, x, pts_pad, nidx_flat)


def _stage3a_body(npts_ref, nx_ref, ctr_ref, kpx_ref, kpy_ref, kpz_ref,
                  ex_ref, kwf_ref, t_ref):
    pts = npts_ref[...]
    ctrk = jnp.broadcast_to(ctr_ref[...][:, None, :], (B3, K, 8)).reshape(R3, 8)
    rel = pts - ctrk
    dx = rel[:, 0:1] - kpx_ref[...]
    dy = rel[:, 1:2] - kpy_ref[...]
    dz = rel[:, 2:3] - kpz_ref[...]
    d = jnp.sqrt(dx * dx + dy * dy + dz * dz)
    infl = jnp.maximum(0.0, 1.0 - d * (1.0 / KP_EXTENT))
    inflr = jnp.dot(infl, ex_ref[...], preferred_element_type=jnp.float32)
    nxb = nx_ref[...]
    nxt = jnp.concatenate([nxb] * N_KP, axis=1)
    q = (inflr * nxt).reshape(B3, K, N_KP * MID)
    agg = jnp.sum(q, axis=1)
    t = jnp.dot(agg, kwf_ref[...], preferred_element_type=jnp.float32)
    t_ref[...] = _leaky(t)


def _stage3a(npts, nx, ctr, kpx, kpy, kpz, ex, kwf):
    return pl.pallas_call(
        _stage3a_body,
        grid=(NBLK3,),
        in_specs=[
            pl.BlockSpec((R3, 8), lambda b: (b, 0)),
            pl.BlockSpec((R3, MID), lambda b: (b, 0)),
            pl.BlockSpec((B3, 8), lambda b: (b, 0)),
            pl.BlockSpec((1, 16), lambda b: (0, 0)),
            pl.BlockSpec((1, 16), lambda b: (0, 0)),
            pl.BlockSpec((1, 16), lambda b: (0, 0)),
            pl.BlockSpec((16, N_KP * MID), lambda b: (0, 0)),
            pl.BlockSpec((N_KP * MID, MID), lambda b: (0, 0)),
        ],
        out_specs=pl.BlockSpec((B3, MID), lambda b: (b, 0)),
        out_shape=jax.ShapeDtypeStruct((N_PAD, MID), jnp.float32),
    )(npts, nx, ctr, kpx, kpy, kpz, ex, kwf)


def _stage3b_body(t_ref, sm_ref, w2_ref, b2_ref, g2_ref, be2_ref,
                  wsc_ref, bsc_ref, gsc_ref, besc_ref, o_ref):
    u = jnp.dot(t_ref[...], w2_ref[...], preferred_element_type=jnp.float32)
    u = u + b2_ref[...]
    m = jnp.mean(u, axis=0, keepdims=True)
    v = jnp.mean((u - m) ** 2, axis=0, keepdims=True)
    u = _leaky((u - m) / jnp.sqrt(v + 1e-5) * g2_ref[...] + be2_ref[...])
    s = jnp.dot(sm_ref[...], wsc_ref[...], preferred_element_type=jnp.float32)
    s = s + bsc_ref[...]
    ms = jnp.mean(s, axis=0, keepdims=True)
    vs = jnp.mean((s - ms) ** 2, axis=0, keepdims=True)
    s = _leaky((s - ms) / jnp.sqrt(vs + 1e-5) * gsc_ref[...] + besc_ref[...])
    o_ref[...] = _leaky(u + s)


def _stage3b(t, sm, W2, b2, g2, be2, Wsc, bsc, gsc, besc):
    r = lambda a: a.reshape(1, OUT_DIM)
    return pl.pallas_call(
        _stage3b_body,
        out_shape=jax.ShapeDtypeStruct((N, OUT_DIM), jnp.float32),
    )(t, sm, W2, r(b2), r(g2), r(be2), Wsc, r(bsc), r(gsc), r(besc))


def kernel(features, points, neighbors, W1, b1, g1, be1, kp, KW,
           W2, b2, g2, be2, Wsc, bsc, gsc, besc):
    features = features.astype(jnp.float32)
    nidx = jnp.pad(neighbors.astype(jnp.int32), ((0, N_PAD - N), (0, 0)))
    nidx_flat = nidx.reshape(N_PAD * K)
    pts_pad = jnp.pad(points, ((0, 0), (0, 5)))
    ctr = jnp.pad(points, ((0, N_PAD - N), (0, 5)))
    kpx = jnp.pad(kp[:, 0], (0, 1)).reshape(1, 16)
    kpy = jnp.pad(kp[:, 1], (0, 1)).reshape(1, 16)
    kpz = jnp.pad(kp[:, 2], (0, 1)).reshape(1, 16)
    kwf = KW.reshape(N_KP * MID, MID)
    ex = (jnp.arange(N_KP * MID) // MID ==
          jnp.arange(16)[:, None]).astype(jnp.float32)

    x = _stage1(features, W1, b1, g1, be1)
    scmax, nx_rows, npts_rows = _stage2(features, x, pts_pad, nidx_flat)
    t = _stage3a(npts_rows, nx_rows, ctr, kpx, kpy, kpz, ex, kwf)
    return _stage3b(t[:N], scmax[:N], W2, b2, g2, be2, Wsc, bsc, gsc, besc)


# speedup vs baseline: 3.4920x; 1.2541x over previous
import functools

import jax
import jax.numpy as jnp
from jax import lax
from jax.experimental import pallas as pl
from jax.experimental.pallas import tpu as pltpu
from jax.experimental.pallas import tpu_sc as plsc

N = 10000
K = 32
IN_DIM = 128
OUT_DIM = 128
MID = 32
N_KP = 15
KP_EXTENT = 0.05
NEG_SLOPE = 0.1

NC = 2
NS = 16
NW = NC * NS
NODES_PW = 320
N_PAD = NW * NODES_PW
CH = 8
ROWS = CH * K
NCHUNK = NODES_PW // CH
IPW = NODES_PW * K // 128

B3 = 64
NBLK3 = N_PAD // B3
R3 = B3 * K


def _leaky(x):
    return jnp.where(x >= 0, x, NEG_SLOPE * x)


def _stage1_body(f_ref, w_ref, b_ref, g_ref, be_ref, x_ref):
    y = jnp.dot(f_ref[...], w_ref[...], preferred_element_type=jnp.float32)
    y = y + b_ref[...]
    m = jnp.mean(y, axis=0, keepdims=True)
    v = jnp.mean((y - m) ** 2, axis=0, keepdims=True)
    y = (y - m) / jnp.sqrt(v + 1e-5)
    x_ref[...] = _leaky(y * g_ref[...] + be_ref[...])


def _stage1(features, W1, b1, g1, be1):
    return pl.pallas_call(
        _stage1_body,
        out_shape=jax.ShapeDtypeStruct((N, MID), jnp.float32),
    )(features, W1, b1.reshape(1, MID), g1.reshape(1, MID), be1.reshape(1, MID))


def _sc_gather_body(feat_hbm, x_hbm, pts_hbm, nidx_hbm,
                    scmax_hbm, nx_hbm, npts_hbm,
                    idx_v, fv, xv, pv, mv, gsem, wsem):
    wid = lax.axis_index("s") * NC + lax.axis_index("c")
    pltpu.sync_copy(nidx_hbm.at[pl.ds(wid * IPW, IPW)], idx_v)

    def gathers(t, b):
        cs = []
        for h in range(2):
            irow = 2 * t + h
            dst = pl.ds(h * 128, 128)
            cs.append(pltpu.make_async_copy(
                feat_hbm.at[idx_v.at[irow]], fv[b].at[dst], gsem[b]))
            cs.append(pltpu.make_async_copy(
                x_hbm.at[idx_v.at[irow]], xv[b].at[dst], gsem[b]))
            cs.append(pltpu.make_async_copy(
                pts_hbm.at[idx_v.at[irow]], pv[b].at[dst], gsem[b]))
        return cs

    def writes(t, b):
        node0 = wid * NODES_PW + t * CH
        row0 = node0 * K
        return [
            pltpu.make_async_copy(xv[b], nx_hbm.at[pl.ds(row0, ROWS)], wsem[b]),
            pltpu.make_async_copy(pv[b], npts_hbm.at[pl.ds(row0, ROWS)], wsem[b]),
            pltpu.make_async_copy(mv[b], scmax_hbm.at[pl.ds(node0, CH)], wsem[b]),
        ]

    def compute(t, b):
        for i in range(CH):
            def kmax(k, acc):
                return tuple(
                    jnp.maximum(acc[j], fv[b][i * K + k, pl.ds(j * 32, 32)])
                    for j in range(4))
            acc0 = tuple(fv[b][i * K, pl.ds(j * 32, 32)] for j in range(4))
            acc = lax.fori_loop(1, K, kmax, acc0)
            for j in range(4):
                mv[b][i, pl.ds(j * 32, 32)] = acc[j]
        for c in writes(t, b):
            c.start()

    for c in gathers(0, 0):
        c.start()

    def body(i, carry):
        tA = 2 * i
        tB = 2 * i + 1
        @pl.when(i > 0)
        def _():
            for c in writes(tB - 2, 1):
                c.wait()
        for c in gathers(tB, 1):
            c.start()
        for c in gathers(tA, 0):
            c.wait()
        compute(tA, 0)
        for c in writes(tA, 0):
            c.wait()

        @pl.when(i < NCHUNK // 2 - 1)
        def _():
            for c in gathers(tA + 2, 0):
                c.start()
        for c in gathers(tB, 1):
            c.wait()
        compute(tB, 1)
        return carry

    lax.fori_loop(0, NCHUNK // 2, body, 0)
    for c in writes(NCHUNK - 1, 1):
        c.wait()


def _stage2(featb, xb, pts_pad, nidx2d):
    mesh = plsc.VectorSubcoreMesh(core_axis_name="c", subcore_axis_name="s")

    def wrapped(feat_hbm, x_hbm, pts_hbm, nidx_hbm, scmax_hbm, nx_hbm,
                npts_hbm, idx_v, fv0, fv1, xv0, xv1, pv0, pv1, mv0, mv1,
                gs0, gs1, ws0, ws1):
        _sc_gather_body(feat_hbm, x_hbm, pts_hbm, nidx_hbm,
                        scmax_hbm, nx_hbm, npts_hbm,
                        idx_v, (fv0, fv1), (xv0, xv1), (pv0, pv1),
                        (mv0, mv1), (gs0, gs1), (ws0, ws1))

    fn = functools.partial(
        pl.kernel, wrapped, mesh=mesh,
        compiler_params=pltpu.CompilerParams(use_tc_tiling_on_sc=False),
        out_type=[
            jax.ShapeDtypeStruct((N_PAD, IN_DIM), jnp.bfloat16),
            jax.ShapeDtypeStruct((N_PAD * K, MID), jnp.bfloat16),
            jax.ShapeDtypeStruct((N_PAD * K, 8), jnp.float32),
        ],
        scratch_types=[
            pltpu.VMEM((IPW, 128), jnp.int32),
            pltpu.VMEM((ROWS, IN_DIM), jnp.bfloat16),
            pltpu.VMEM((ROWS, IN_DIM), jnp.bfloat16),
            pltpu.VMEM((ROWS, MID), jnp.bfloat16),
            pltpu.VMEM((ROWS, MID), jnp.bfloat16),
            pltpu.VMEM((ROWS, 8), jnp.float32),
            pltpu.VMEM((ROWS, 8), jnp.float32),
            pltpu.VMEM((CH, IN_DIM), jnp.bfloat16),
            pltpu.VMEM((CH, IN_DIM), jnp.bfloat16),
            pltpu.SemaphoreType.DMA,
            pltpu.SemaphoreType.DMA,
            pltpu.SemaphoreType.DMA,
            pltpu.SemaphoreType.DMA,
        ],
    )()
    return fn(featb, xb, pts_pad, nidx2d)


def _stage3a_body(npts_ref, nx_ref, ctr_ref, kpx_ref, kpy_ref, kpz_ref,
                  ex_ref, kwf_ref, t_ref):
    pts = npts_ref[...]
    ctrk = jnp.broadcast_to(ctr_ref[...][:, None, :], (B3, K, 8)).reshape(R3, 8)
    rel = pts - ctrk
    dx = rel[:, 0:1] - kpx_ref[...]
    dy = rel[:, 1:2] - kpy_ref[...]
    dz = rel[:, 2:3] - kpz_ref[...]
    d = jnp.sqrt(dx * dx + dy * dy + dz * dz)
    infl = jnp.maximum(0.0, 1.0 - d * (1.0 / KP_EXTENT))
    inflr = jnp.dot(infl, ex_ref[...], preferred_element_type=jnp.float32)
    nxb = nx_ref[...].astype(jnp.float32)
    nxt = jnp.concatenate([nxb] * N_KP, axis=1)
    q = (inflr * nxt).reshape(B3, K, N_KP * MID)
    agg = jnp.sum(q, axis=1)
    t = jnp.dot(agg, kwf_ref[...], preferred_element_type=jnp.float32)
    t_ref[...] = _leaky(t)


def _stage3a(npts, nx, ctr, kpx, kpy, kpz, ex, kwf):
    return pl.pallas_call(
        _stage3a_body,
        grid=(NBLK3,),
        in_specs=[
            pl.BlockSpec((R3, 8), lambda b: (b, 0)),
            pl.BlockSpec((R3, MID), lambda b: (b, 0)),
            pl.BlockSpec((B3, 8), lambda b: (b, 0)),
            pl.BlockSpec((1, 16), lambda b: (0, 0)),
            pl.BlockSpec((1, 16), lambda b: (0, 0)),
            pl.BlockSpec((1, 16), lambda b: (0, 0)),
            pl.BlockSpec((16, N_KP * MID), lambda b: (0, 0)),
            pl.BlockSpec((N_KP * MID, MID), lambda b: (0, 0)),
        ],
        out_specs=pl.BlockSpec((B3, MID), lambda b: (b, 0)),
        out_shape=jax.ShapeDtypeStruct((N_PAD, MID), jnp.float32),
    )(npts, nx, ctr, kpx, kpy, kpz, ex, kwf)


def _stage3b_body(t_ref, sm_ref, w2_ref, b2_ref, g2_ref, be2_ref,
                  wsc_ref, bsc_ref, gsc_ref, besc_ref, o_ref):
    u = jnp.dot(t_ref[...], w2_ref[...], preferred_element_type=jnp.float32)
    u = u + b2_ref[...]
    m = jnp.mean(u, axis=0, keepdims=True)
    v = jnp.mean((u - m) ** 2, axis=0, keepdims=True)
    u = _leaky((u - m) / jnp.sqrt(v + 1e-5) * g2_ref[...] + be2_ref[...])
    s = jnp.dot(sm_ref[...].astype(jnp.float32), wsc_ref[...],
                preferred_element_type=jnp.float32)
    s = s + bsc_ref[...]
    ms = jnp.mean(s, axis=0, keepdims=True)
    vs = jnp.mean((s - ms) ** 2, axis=0, keepdims=True)
    s = _leaky((s - ms) / jnp.sqrt(vs + 1e-5) * gsc_ref[...] + besc_ref[...])
    o_ref[...] = _leaky(u + s)


def _stage3b(t, sm, W2, b2, g2, be2, Wsc, bsc, gsc, besc):
    r = lambda a: a.reshape(1, OUT_DIM)
    return pl.pallas_call(
        _stage3b_body,
        out_shape=jax.ShapeDtypeStruct((N, OUT_DIM), jnp.float32),
    )(t, sm, W2, r(b2), r(g2), r(be2), Wsc, r(bsc), r(gsc), r(besc))


def kernel(features, points, neighbors, W1, b1, g1, be1, kp, KW,
           W2, b2, g2, be2, Wsc, bsc, gsc, besc):
    features = features.astype(jnp.float32)
    nidx = jnp.pad(neighbors.astype(jnp.int32), ((0, N_PAD - N), (0, 0)))
    nidx_flat = nidx.reshape(N_PAD * K)
    pts_pad = jnp.pad(points, ((0, 0), (0, 5)))
    ctr = jnp.pad(points, ((0, N_PAD - N), (0, 5)))
    kpx = jnp.pad(kp[:, 0], (0, 1)).reshape(1, 16)
    kpy = jnp.pad(kp[:, 1], (0, 1)).reshape(1, 16)
    kpz = jnp.pad(kp[:, 2], (0, 1)).reshape(1, 16)
    kwf = KW.reshape(N_KP * MID, MID)
    ex = (jnp.arange(N_KP * MID) // MID ==
          jnp.arange(16)[:, None]).astype(jnp.float32)

    x = _stage1(features, W1, b1, g1, be1)
    featb = features.astype(jnp.bfloat16)
    xb = x.astype(jnp.bfloat16)
    nidx2d = nidx_flat.reshape(N_PAD * K // 128, 128)
    scmax, nx_rows, npts_rows = _stage2(featb, xb, pts_pad, nidx2d)
    t = _stage3a(npts_rows, nx_rows, ctr, kpx, kpy, kpz, ex, kwf)
    return _stage3b(t[:N], scmax[:N], W2, b2, g2, be2, Wsc, bsc, gsc, besc)


# speedup vs baseline: 3.5395x; 1.0136x over previous
import functools

import jax
import jax.numpy as jnp
from jax import lax
from jax.experimental import pallas as pl
from jax.experimental.pallas import tpu as pltpu
from jax.experimental.pallas import tpu_sc as plsc

N = 10000
K = 32
IN_DIM = 128
OUT_DIM = 128
MID = 32
N_KP = 15
KP_EXTENT = 0.05
NEG_SLOPE = 0.1

NC = 2
NS = 16
NW = NC * NS
NODES_PW = 320
N_PAD = NW * NODES_PW
CH = 8
ROWS = CH * K
NCHUNK = NODES_PW // CH
IPW = NODES_PW * K // 128

B3 = 128
NBLK3 = N_PAD // B3
R3 = B3 * K


def _leaky(x):
    return jnp.where(x >= 0, x, NEG_SLOPE * x)


def _stage1_body(f_ref, w_ref, b_ref, g_ref, be_ref, x_ref):
    y = jnp.dot(f_ref[...], w_ref[...], preferred_element_type=jnp.float32)
    y = y + b_ref[...]
    m = jnp.mean(y, axis=0, keepdims=True)
    v = jnp.mean((y - m) ** 2, axis=0, keepdims=True)
    y = (y - m) / jnp.sqrt(v + 1e-5)
    x_ref[...] = _leaky(y * g_ref[...] + be_ref[...])


def _stage1(features, W1, b1, g1, be1):
    return pl.pallas_call(
        _stage1_body,
        out_shape=jax.ShapeDtypeStruct((N, MID), jnp.float32),
    )(features, W1, b1.reshape(1, MID), g1.reshape(1, MID), be1.reshape(1, MID))


def _sc_gather_body(feat_hbm, x_hbm, pts_hbm, nidx_hbm,
                    scmax_hbm, nx_hbm, npts_hbm,
                    idx_v, fv, xv, pv, mv, gsem, wsem):
    wid = lax.axis_index("s") * NC + lax.axis_index("c")
    pltpu.sync_copy(nidx_hbm.at[pl.ds(wid * IPW, IPW)], idx_v)

    def gathers(t, b):
        cs = []
        for h in range(2):
            irow = 2 * t + h
            dst = pl.ds(h * 128, 128)
            cs.append(pltpu.make_async_copy(
                feat_hbm.at[idx_v.at[irow]], fv[b].at[dst], gsem[b]))
            cs.append(pltpu.make_async_copy(
                x_hbm.at[idx_v.at[irow]], xv[b].at[dst], gsem[b]))
            cs.append(pltpu.make_async_copy(
                pts_hbm.at[idx_v.at[irow]], pv[b].at[dst], gsem[b]))
        return cs

    def writes(t, b):
        node0 = wid * NODES_PW + t * CH
        row0 = node0 * K
        return [
            pltpu.make_async_copy(xv[b], nx_hbm.at[pl.ds(row0, ROWS)], wsem[b]),
            pltpu.make_async_copy(pv[b], npts_hbm.at[pl.ds(row0, ROWS)], wsem[b]),
            pltpu.make_async_copy(mv[b], scmax_hbm.at[pl.ds(node0, CH)], wsem[b]),
        ]

    def compute(t, b):
        for i in range(CH):
            def kmax(k, acc):
                return tuple(
                    jnp.maximum(acc[j], fv[b][i * K + k, pl.ds(j * 32, 32)])
                    for j in range(4))
            acc0 = tuple(fv[b][i * K, pl.ds(j * 32, 32)] for j in range(4))
            acc = lax.fori_loop(1, K, kmax, acc0)
            for j in range(4):
                mv[b][i, pl.ds(j * 32, 32)] = acc[j]
        for c in writes(t, b):
            c.start()

    for c in gathers(0, 0):
        c.start()

    def body(i, carry):
        tA = 2 * i
        tB = 2 * i + 1
        @pl.when(i > 0)
        def _():
            for c in writes(tB - 2, 1):
                c.wait()
        for c in gathers(tB, 1):
            c.start()
        for c in gathers(tA, 0):
            c.wait()
        compute(tA, 0)
        for c in writes(tA, 0):
            c.wait()

        @pl.when(i < NCHUNK // 2 - 1)
        def _():
            for c in gathers(tA + 2, 0):
                c.start()
        for c in gathers(tB, 1):
            c.wait()
        compute(tB, 1)
        return carry

    lax.fori_loop(0, NCHUNK // 2, body, 0)
    for c in writes(NCHUNK - 1, 1):
        c.wait()


def _stage2(featb, xb, pts_pad, nidx2d):
    mesh = plsc.VectorSubcoreMesh(core_axis_name="c", subcore_axis_name="s")

    def wrapped(feat_hbm, x_hbm, pts_hbm, nidx_hbm, scmax_hbm, nx_hbm,
                npts_hbm, idx_v, fv0, fv1, xv0, xv1, pv0, pv1, mv0, mv1,
                gs0, gs1, ws0, ws1):
        _sc_gather_body(feat_hbm, x_hbm, pts_hbm, nidx_hbm,
                        scmax_hbm, nx_hbm, npts_hbm,
                        idx_v, (fv0, fv1), (xv0, xv1), (pv0, pv1),
                        (mv0, mv1), (gs0, gs1), (ws0, ws1))

    fn = functools.partial(
        pl.kernel, wrapped, mesh=mesh,
        compiler_params=pltpu.CompilerParams(use_tc_tiling_on_sc=False),
        out_type=[
            jax.ShapeDtypeStruct((N_PAD, IN_DIM), jnp.bfloat16),
            jax.ShapeDtypeStruct((N_PAD * K, MID), jnp.bfloat16),
            jax.ShapeDtypeStruct((N_PAD * K, 8), jnp.float32),
        ],
        scratch_types=[
            pltpu.VMEM((IPW, 128), jnp.int32),
            pltpu.VMEM((ROWS, IN_DIM), jnp.bfloat16),
            pltpu.VMEM((ROWS, IN_DIM), jnp.bfloat16),
            pltpu.VMEM((ROWS, MID), jnp.bfloat16),
            pltpu.VMEM((ROWS, MID), jnp.bfloat16),
            pltpu.VMEM((ROWS, 8), jnp.float32),
            pltpu.VMEM((ROWS, 8), jnp.float32),
            pltpu.VMEM((CH, IN_DIM), jnp.bfloat16),
            pltpu.VMEM((CH, IN_DIM), jnp.bfloat16),
            pltpu.SemaphoreType.DMA,
            pltpu.SemaphoreType.DMA,
            pltpu.SemaphoreType.DMA,
            pltpu.SemaphoreType.DMA,
        ],
    )()
    return fn(featb, xb, pts_pad, nidx2d)


def _stage3a_body(npts_ref, nx_ref, ctr_ref, kpx_ref, kpy_ref, kpz_ref,
                  ex_ref, kwf_ref, t_ref):
    pts = npts_ref[...]
    ctrk = jnp.broadcast_to(ctr_ref[...][:, None, :], (B3, K, 8)).reshape(R3, 8)
    rel = pts - ctrk
    dx = rel[:, 0:1] - kpx_ref[...]
    dy = rel[:, 1:2] - kpy_ref[...]
    dz = rel[:, 2:3] - kpz_ref[...]
    d = jnp.sqrt(dx * dx + dy * dy + dz * dz)
    infl = jnp.maximum(0.0, 1.0 - d * (1.0 / KP_EXTENT))
    inflr = jnp.dot(infl, ex_ref[...],
                    preferred_element_type=jnp.float32).astype(jnp.bfloat16)
    nxb = nx_ref[...]
    nxt = jnp.concatenate([nxb] * N_KP, axis=1)
    q = (inflr * nxt).reshape(B3, K, N_KP * MID)
    agg = jnp.sum(q, axis=1, dtype=jnp.float32)
    t = jnp.dot(agg, kwf_ref[...], preferred_element_type=jnp.float32)
    t_ref[...] = _leaky(t)


def _stage3a(npts, nx, ctr, kpx, kpy, kpz, ex, kwf):
    return pl.pallas_call(
        _stage3a_body,
        grid=(NBLK3,),
        in_specs=[
            pl.BlockSpec((R3, 8), lambda b: (b, 0)),
            pl.BlockSpec((R3, MID), lambda b: (b, 0)),
            pl.BlockSpec((B3, 8), lambda b: (b, 0)),
            pl.BlockSpec((1, 16), lambda b: (0, 0)),
            pl.BlockSpec((1, 16), lambda b: (0, 0)),
            pl.BlockSpec((1, 16), lambda b: (0, 0)),
            pl.BlockSpec((16, N_KP * MID), lambda b: (0, 0)),
            pl.BlockSpec((N_KP * MID, MID), lambda b: (0, 0)),
        ],
        out_specs=pl.BlockSpec((B3, MID), lambda b: (b, 0)),
        out_shape=jax.ShapeDtypeStruct((N_PAD, MID), jnp.float32),
    )(npts, nx, ctr, kpx, kpy, kpz, ex, kwf)


def _stage3b_body(t_ref, sm_ref, w2_ref, b2_ref, g2_ref, be2_ref,
                  wsc_ref, bsc_ref, gsc_ref, besc_ref, o_ref):
    u = jnp.dot(t_ref[...], w2_ref[...], preferred_element_type=jnp.float32)
    u = u + b2_ref[...]
    m = jnp.mean(u, axis=0, keepdims=True)
    v = jnp.mean((u - m) ** 2, axis=0, keepdims=True)
    u = _leaky((u - m) / jnp.sqrt(v + 1e-5) * g2_ref[...] + be2_ref[...])
    s = jnp.dot(sm_ref[...].astype(jnp.float32), wsc_ref[...],
                preferred_element_type=jnp.float32)
    s = s + bsc_ref[...]
    ms = jnp.mean(s, axis=0, keepdims=True)
    vs = jnp.mean((s - ms) ** 2, axis=0, keepdims=True)
    s = _leaky((s - ms) / jnp.sqrt(vs + 1e-5) * gsc_ref[...] + besc_ref[...])
    o_ref[...] = _leaky(u + s)


def _stage3b(t, sm, W2, b2, g2, be2, Wsc, bsc, gsc, besc):
    r = lambda a: a.reshape(1, OUT_DIM)
    return pl.pallas_call(
        _stage3b_body,
        out_shape=jax.ShapeDtypeStruct((N, OUT_DIM), jnp.float32),
    )(t, sm, W2, r(b2), r(g2), r(be2), Wsc, r(bsc), r(gsc), r(besc))


def kernel(features, points, neighbors, W1, b1, g1, be1, kp, KW,
           W2, b2, g2, be2, Wsc, bsc, gsc, besc):
    features = features.astype(jnp.float32)
    nidx = jnp.pad(neighbors.astype(jnp.int32), ((0, N_PAD - N), (0, 0)))
    nidx_flat = nidx.reshape(N_PAD * K)
    pts_pad = jnp.pad(points, ((0, 0), (0, 5)))
    ctr = jnp.pad(points, ((0, N_PAD - N), (0, 5)))
    kpx = jnp.pad(kp[:, 0], (0, 1)).reshape(1, 16)
    kpy = jnp.pad(kp[:, 1], (0, 1)).reshape(1, 16)
    kpz = jnp.pad(kp[:, 2], (0, 1)).reshape(1, 16)
    kwf = KW.reshape(N_KP * MID, MID)
    ex = (jnp.arange(N_KP * MID) // MID ==
          jnp.arange(16)[:, None]).astype(jnp.float32)

    x = _stage1(features, W1, b1, g1, be1)
    featb = features.astype(jnp.bfloat16)
    xb = x.astype(jnp.bfloat16)
    nidx2d = nidx_flat.reshape(N_PAD * K // 128, 128)
    scmax, nx_rows, npts_rows = _stage2(featb, xb, pts_pad, nidx2d)
    t = _stage3a(npts_rows, nx_rows, ctr, kpx, kpy, kpz, ex, kwf)
    return _stage3b(t[:N], scmax[:N], W2, b2, g2, be2, Wsc, bsc, gsc, besc)


# speedup vs baseline: 4.0473x; 1.1435x over previous
import functools

import jax
import jax.numpy as jnp
from jax import lax
from jax.experimental import pallas as pl
from jax.experimental.pallas import tpu as pltpu
from jax.experimental.pallas import tpu_sc as plsc

N = 10000
K = 32
IN_DIM = 128
OUT_DIM = 128
MID = 32
N_KP = 15
KP_EXTENT = 0.05
NEG_SLOPE = 0.1

NC = 2
NS = 16
NW = NC * NS
NODES_PW = 320
N_PAD = NW * NODES_PW
CH = 8
ROWS = CH * K
NCHUNK = NODES_PW // CH
IPW = NODES_PW * K // 128

B3 = 128
NBLK3 = N_PAD // B3
R3 = B3 * K


def _leaky(x):
    return jnp.where(x >= 0, x, NEG_SLOPE * x)


def _stage1_body(f_ref, w_ref, b_ref, g_ref, be_ref, x_ref):
    y = jnp.dot(f_ref[...], w_ref[...], preferred_element_type=jnp.float32)
    y = y + b_ref[...]
    m = jnp.mean(y, axis=0, keepdims=True)
    v = jnp.mean((y - m) ** 2, axis=0, keepdims=True)
    y = (y - m) / jnp.sqrt(v + 1e-5)
    x_ref[...] = _leaky(y * g_ref[...] + be_ref[...])


def _stage1(features, W1, b1, g1, be1):
    return pl.pallas_call(
        _stage1_body,
        out_shape=jax.ShapeDtypeStruct((N, MID), jnp.float32),
    )(features, W1, b1.reshape(1, MID), g1.reshape(1, MID), be1.reshape(1, MID))


def _sc_gather_body(feat_hbm, x_hbm, pts_hbm, nidx_hbm,
                    scmax_hbm, nx_hbm, npts_hbm,
                    idx_v, fv, xv, pv, mv, gsem, wsem):
    wid = lax.axis_index("s") * NC + lax.axis_index("c")
    pltpu.sync_copy(nidx_hbm.at[pl.ds(wid * IPW, IPW)], idx_v)

    def gathers(t, b):
        cs = []
        for h in range(2):
            irow = 2 * t + h
            dst = pl.ds(h * 128, 128)
            cs.append(pltpu.make_async_copy(
                feat_hbm.at[idx_v.at[irow]], fv[b].at[dst], gsem[b]))
            cs.append(pltpu.make_async_copy(
                x_hbm.at[idx_v.at[irow]], xv[b].at[dst], gsem[b]))
            cs.append(pltpu.make_async_copy(
                pts_hbm.at[idx_v.at[irow]], pv[b].at[dst], gsem[b]))
        return cs

    def writes(t, b):
        node0 = wid * NODES_PW + t * CH
        row0 = node0 * K
        return [
            pltpu.make_async_copy(xv[b], nx_hbm.at[pl.ds(row0, ROWS)], wsem[b]),
            pltpu.make_async_copy(pv[b], npts_hbm.at[pl.ds(row0, ROWS)], wsem[b]),
            pltpu.make_async_copy(mv[b], scmax_hbm.at[pl.ds(node0, CH)], wsem[b]),
        ]

    def compute(t, b):
        for i in range(CH):
            def kmax(k, acc):
                return tuple(
                    jnp.maximum(acc[j], fv[b][i * K + k, pl.ds(j * 32, 32)])
                    for j in range(4))
            acc0 = tuple(fv[b][i * K, pl.ds(j * 32, 32)] for j in range(4))
            acc = lax.fori_loop(1, K, kmax, acc0, unroll=4)
            for j in range(4):
                mv[b][i, pl.ds(j * 32, 32)] = acc[j]
        for c in writes(t, b):
            c.start()

    for c in gathers(0, 0):
        c.start()

    def body(i, carry):
        tA = 2 * i
        tB = 2 * i + 1
        @pl.when(i > 0)
        def _():
            for c in writes(tB - 2, 1):
                c.wait()
        for c in gathers(tB, 1):
            c.start()
        for c in gathers(tA, 0):
            c.wait()
        compute(tA, 0)
        for c in writes(tA, 0):
            c.wait()

        @pl.when(i < NCHUNK // 2 - 1)
        def _():
            for c in gathers(tA + 2, 0):
                c.start()
        for c in gathers(tB, 1):
            c.wait()
        compute(tB, 1)
        return carry

    lax.fori_loop(0, NCHUNK // 2, body, 0)
    for c in writes(NCHUNK - 1, 1):
        c.wait()


def _stage2(featb, xb, pts_pad, nidx2d):
    mesh = plsc.VectorSubcoreMesh(core_axis_name="c", subcore_axis_name="s")

    def wrapped(feat_hbm, x_hbm, pts_hbm, nidx_hbm, scmax_hbm, nx_hbm,
                npts_hbm, idx_v, fv0, fv1, xv0, xv1, pv0, pv1, mv0, mv1,
                gs0, gs1, ws0, ws1):
        _sc_gather_body(feat_hbm, x_hbm, pts_hbm, nidx_hbm,
                        scmax_hbm, nx_hbm, npts_hbm,
                        idx_v, (fv0, fv1), (xv0, xv1), (pv0, pv1),
                        (mv0, mv1), (gs0, gs1), (ws0, ws1))

    fn = functools.partial(
        pl.kernel, wrapped, mesh=mesh,
        compiler_params=pltpu.CompilerParams(use_tc_tiling_on_sc=False),
        out_type=[
            jax.ShapeDtypeStruct((N_PAD, IN_DIM), jnp.bfloat16),
            jax.ShapeDtypeStruct((N_PAD * K, MID), jnp.bfloat16),
            jax.ShapeDtypeStruct((N_PAD * K, 8), jnp.float32),
        ],
        scratch_types=[
            pltpu.VMEM((IPW, 128), jnp.int32),
            pltpu.VMEM((ROWS, IN_DIM), jnp.bfloat16),
            pltpu.VMEM((ROWS, IN_DIM), jnp.bfloat16),
            pltpu.VMEM((ROWS, MID), jnp.bfloat16),
            pltpu.VMEM((ROWS, MID), jnp.bfloat16),
            pltpu.VMEM((ROWS, 8), jnp.float32),
            pltpu.VMEM((ROWS, 8), jnp.float32),
            pltpu.VMEM((CH, IN_DIM), jnp.bfloat16),
            pltpu.VMEM((CH, IN_DIM), jnp.bfloat16),
            pltpu.SemaphoreType.DMA,
            pltpu.SemaphoreType.DMA,
            pltpu.SemaphoreType.DMA,
            pltpu.SemaphoreType.DMA,
        ],
    )()
    return fn(featb, xb, pts_pad, nidx2d)


def _stage3a_body(npts_ref, nx_ref, ctr_ref, dmat_ref, ksq_ref,
                  ex_ref, kwf_ref, t_ref):
    pts = npts_ref[...]
    ctrk = jnp.broadcast_to(ctr_ref[...][:, None, :], (B3, K, 8)).reshape(R3, 8)
    rel = pts - ctrk
    cat = jnp.concatenate([rel, rel * rel], axis=1)
    m16 = jnp.dot(cat, dmat_ref[...], preferred_element_type=jnp.float32)
    d2 = jnp.maximum(m16 + ksq_ref[...], 0.0)
    infl = jnp.maximum(0.0, 1.0 - jnp.sqrt(d2) * (1.0 / KP_EXTENT))
    inflr = jnp.dot(infl, ex_ref[...],
                    preferred_element_type=jnp.float32).astype(jnp.bfloat16)
    nxb = nx_ref[...]
    nxt = jnp.concatenate([nxb] * N_KP, axis=1)
    q = (inflr * nxt).reshape(B3, K, N_KP * MID)
    agg = jnp.sum(q, axis=1, dtype=jnp.float32)
    t = jnp.dot(agg, kwf_ref[...], preferred_element_type=jnp.float32)
    t_ref[...] = _leaky(t)


def _stage3a(npts, nx, ctr, dmat, ksq, ex, kwf):
    return pl.pallas_call(
        _stage3a_body,
        grid=(NBLK3,),
        in_specs=[
            pl.BlockSpec((R3, 8), lambda b: (b, 0)),
            pl.BlockSpec((R3, MID), lambda b: (b, 0)),
            pl.BlockSpec((B3, 8), lambda b: (b, 0)),
            pl.BlockSpec((16, 16), lambda b: (0, 0)),
            pl.BlockSpec((1, 16), lambda b: (0, 0)),
            pl.BlockSpec((16, N_KP * MID), lambda b: (0, 0)),
            pl.BlockSpec((N_KP * MID, MID), lambda b: (0, 0)),
        ],
        out_specs=pl.BlockSpec((B3, MID), lambda b: (b, 0)),
        out_shape=jax.ShapeDtypeStruct((N_PAD, MID), jnp.float32),
    )(npts, nx, ctr, dmat, ksq, ex, kwf)


def _stage3b_body(t_ref, sm_ref, w2_ref, b2_ref, g2_ref, be2_ref,
                  wsc_ref, bsc_ref, gsc_ref, besc_ref, o_ref):
    u = jnp.dot(t_ref[...], w2_ref[...], preferred_element_type=jnp.float32)
    u = u + b2_ref[...]
    m = jnp.mean(u, axis=0, keepdims=True)
    v = jnp.mean((u - m) ** 2, axis=0, keepdims=True)
    u = _leaky((u - m) / jnp.sqrt(v + 1e-5) * g2_ref[...] + be2_ref[...])
    s = jnp.dot(sm_ref[...].astype(jnp.float32), wsc_ref[...],
                preferred_element_type=jnp.float32)
    s = s + bsc_ref[...]
    ms = jnp.mean(s, axis=0, keepdims=True)
    vs = jnp.mean((s - ms) ** 2, axis=0, keepdims=True)
    s = _leaky((s - ms) / jnp.sqrt(vs + 1e-5) * gsc_ref[...] + besc_ref[...])
    o_ref[...] = _leaky(u + s)


def _stage3b(t, sm, W2, b2, g2, be2, Wsc, bsc, gsc, besc):
    r = lambda a: a.reshape(1, OUT_DIM)
    return pl.pallas_call(
        _stage3b_body,
        out_shape=jax.ShapeDtypeStruct((N, OUT_DIM), jnp.float32),
    )(t, sm, W2, r(b2), r(g2), r(be2), Wsc, r(bsc), r(gsc), r(besc))


def kernel(features, points, neighbors, W1, b1, g1, be1, kp, KW,
           W2, b2, g2, be2, Wsc, bsc, gsc, besc):
    features = features.astype(jnp.float32)
    nidx = jnp.pad(neighbors.astype(jnp.int32), ((0, N_PAD - N), (0, 0)))
    nidx_flat = nidx.reshape(N_PAD * K)
    pts_pad = jnp.pad(points, ((0, 0), (0, 5)))
    ctr = jnp.pad(points, ((0, N_PAD - N), (0, 5)))
    kp_ext = jnp.pad(kp, ((0, 1), (0, 5)))
    dmat = jnp.concatenate([-2.0 * kp_ext.T,
                            jnp.ones((8, 16), jnp.float32)], axis=0)
    ksq = (jnp.sum(kp_ext ** 2, axis=1) +
           (jnp.arange(16) == 15) * 1e6).reshape(1, 16).astype(jnp.float32)
    kwf = KW.reshape(N_KP * MID, MID)
    ex = (jnp.arange(N_KP * MID) // MID ==
          jnp.arange(16)[:, None]).astype(jnp.float32)

    x = _stage1(features, W1, b1, g1, be1)
    featb = features.astype(jnp.bfloat16)
    xb = x.astype(jnp.bfloat16)
    nidx2d = nidx_flat.reshape(N_PAD * K // 128, 128)
    scmax, nx_rows, npts_rows = _stage2(featb, xb, pts_pad, nidx2d)
    t = _stage3a(npts_rows, nx_rows, ctr, dmat, ksq, ex, kwf)
    return _stage3b(t[:N], scmax[:N], W2, b2, g2, be2, Wsc, bsc, gsc, besc)
